# baseline (speedup 1.0000x reference)
import functools

import jax
import jax.numpy as jnp
from jax import lax
from jax.experimental import pallas as pl
from jax.experimental.pallas import tpu as pltpu
from jax.experimental.pallas import tpu_sc as plsc

B = 128
R = 2048
C = 128
TOPK_K = 32
INV_N = 1.0 / (R * C)

NBUF = 6
L = 16


HB = B // 2


def _mse_hybrid_body(o_blk, l_blk, o_hbm, l_hbm, out_ref, obuf, lbuf, osem, lsem):
    i = pl.program_id(0)

    def _copies(c, slot):
        for arr, buf, sem in ((o_hbm, obuf, osem), (l_hbm, lbuf, lsem)):
            yield pltpu.make_async_copy(
                arr.at[HB + c], buf.at[slot], sem.at[slot])

    def start(c, slot):
        for cp in _copies(c, slot):
            cp.start()

    @pl.when(i == 0)
    def _():
        for c in range(NBUF):
            start(c, c)

    d = o_blk[0] - l_blk[0]
    out_ref[i] = jnp.sum(d * d) * INV_N

    slot = lax.rem(i, NBUF)
    for cp in _copies(i, slot):
        cp.wait()
    d2 = obuf[slot] - lbuf[slot]
    out_ref[HB + i] = jnp.sum(d2 * d2) * INV_N

    @pl.when(i + NBUF < HB)
    def _():
        start(i + NBUF, slot)


def _tc_losses(output, label):
    return pl.pallas_call(
        _mse_hybrid_body,
        grid=(HB,),
        in_specs=[
            pl.BlockSpec((1, R, C), lambda i: (i, 0, 0)),
            pl.BlockSpec((1, R, C), lambda i: (i, 0, 0)),
            pl.BlockSpec(memory_space=pl.ANY),
            pl.BlockSpec(memory_space=pl.ANY),
        ],
        out_specs=pl.BlockSpec(memory_space=pltpu.SMEM),
        out_shape=jax.ShapeDtypeStruct((B,), jnp.float32),
        scratch_shapes=[
            pltpu.VMEM((NBUF, R, C), jnp.float32),
            pltpu.VMEM((NBUF, R, C), jnp.float32),
            pltpu.SemaphoreType.DMA((NBUF,)),
            pltpu.SemaphoreType.DMA((NBUF,)),
        ],
    )(output, label, output, label)


@functools.partial(
    pl.kernel,
    out_type=jax.ShapeDtypeStruct((TOPK_K,), jnp.float32),
    mesh=plsc.VectorSubcoreMesh(core_axis_name="c", subcore_axis_name="s"),
    compiler_params=pltpu.CompilerParams(needs_layout_passes=False),
    scratch_types=[
        pltpu.VMEM((B,), jnp.float32),
        pltpu.VMEM((TOPK_K,), jnp.float32),
    ],
)
def _topk_sc(losses_hbm, out_hbm, vals_v, out_v):
    cid = lax.axis_index("c")
    sid = lax.axis_index("s")

    @pl.when(jnp.logical_and(cid == 0, sid == 0))
    def _():
        pltpu.sync_copy(losses_hbm, vals_v)
        lane = lax.iota(jnp.int32, L)
        nv = B // L
        v = [vals_v[pl.ds(j * L, L)] for j in range(nv)]
        idx = [lane + j * L for j in range(nv)]
        big = jnp.int32(2 ** 30)
        outs = [jnp.zeros((L,), jnp.float32) for _ in range(TOPK_K // L)]
        for r in range(TOPK_K):
            t = v[0]
            for j in range(1, nv):
                t = jnp.maximum(t, v[j])
            m = jnp.max(t)
            c = jnp.where(v[0] == m, idx[0], big)
            for j in range(1, nv):
                c = jnp.minimum(c, jnp.where(v[j] == m, idx[j], big))
            mi = jnp.min(c)
            for j in range(nv):
                v[j] = jnp.where(idx[j] == mi, jnp.float32(-1.0), v[j])
            q, p = divmod(r, L)
            outs[q] = jnp.where(lane == p, m, outs[q])
        for q in range(TOPK_K // L):
            out_v[pl.ds(q * L, L)] = outs[q]
        pltpu.sync_copy(out_v, out_hbm)


def kernel(output, label):
    return _topk_sc(_tc_losses(output, label))


# speedup vs baseline: 1.1918x; 1.1918x over previous
import functools

import jax
import jax.numpy as jnp
from jax import lax
from jax.experimental import pallas as pl
from jax.experimental.pallas import tpu as pltpu
from jax.experimental.pallas import tpu_sc as plsc

B = 128
R = 2048
C = 128
TOPK_K = 32
INV_N = 1.0 / (R * C)

NBUF = 6
L = 16


def _mse_manual_body(o_hbm, l_hbm, out_ref, obuf, lbuf, osem, lsem):
    H = R // 2

    def _copies(c, slot):
        for arr, buf, sem in ((o_hbm, obuf, osem), (l_hbm, lbuf, lsem)):
            for h in range(2):
                yield pltpu.make_async_copy(
                    arr.at[c, pl.ds(h * H, H), :],
                    buf.at[slot, pl.ds(h * H, H), :],
                    sem.at[slot])

    def start(c, slot):
        for cp in _copies(c, slot):
            cp.start()

    for c in range(NBUF):
        start(c, c)

    def chunk_body(i, carry):
        slot = lax.rem(i, NBUF)
        for cp in _copies(i, slot):
            cp.wait()
        d = obuf[slot] - lbuf[slot]
        out_ref[i] = jnp.sum(d * d) * INV_N

        @pl.when(i + NBUF < B)
        def _():
            start(i + NBUF, slot)

        return carry

    lax.fori_loop(0, B, chunk_body, 0)


def _tc_losses(output, label):
    return pl.pallas_call(
        _mse_manual_body,
        in_specs=[
            pl.BlockSpec(memory_space=pl.ANY),
            pl.BlockSpec(memory_space=pl.ANY),
        ],
        out_specs=pl.BlockSpec(memory_space=pltpu.SMEM),
        out_shape=jax.ShapeDtypeStruct((B,), jnp.float32),
        scratch_shapes=[
            pltpu.VMEM((NBUF, R, C), jnp.float32),
            pltpu.VMEM((NBUF, R, C), jnp.float32),
            pltpu.SemaphoreType.DMA((NBUF,)),
            pltpu.SemaphoreType.DMA((NBUF,)),
        ],
    )(output, label)


@functools.partial(
    pl.kernel,
    out_type=jax.ShapeDtypeStruct((TOPK_K,), jnp.float32),
    mesh=plsc.VectorSubcoreMesh(core_axis_name="c", subcore_axis_name="s"),
    compiler_params=pltpu.CompilerParams(needs_layout_passes=False),
    scratch_types=[
        pltpu.VMEM((B,), jnp.float32),
        pltpu.VMEM((TOPK_K,), jnp.float32),
    ],
)
def _topk_sc(losses_hbm, out_hbm, vals_v, out_v):
    cid = lax.axis_index("c")
    sid = lax.axis_index("s")

    @pl.when(jnp.logical_and(cid == 0, sid == 0))
    def _():
        pltpu.sync_copy(losses_hbm, vals_v)
        lane = lax.iota(jnp.int32, L)
        nv = B // L
        v = [vals_v[pl.ds(j * L, L)] for j in range(nv)]
        idx = [lane + j * L for j in range(nv)]
        big = jnp.int32(2 ** 30)
        outs = [jnp.zeros((L,), jnp.float32) for _ in range(TOPK_K // L)]
        for r in range(TOPK_K):
            t = v[0]
            for j in range(1, nv):
                t = jnp.maximum(t, v[j])
            m = jnp.max(t)
            c = jnp.where(v[0] == m, idx[0], big)
            for j in range(1, nv):
                c = jnp.minimum(c, jnp.where(v[j] == m, idx[j], big))
            mi = jnp.min(c)
            for j in range(nv):
                v[j] = jnp.where(idx[j] == mi, jnp.float32(-1.0), v[j])
            q, p = divmod(r, L)
            outs[q] = jnp.where(lane == p, m, outs[q])
        for q in range(TOPK_K // L):
            out_v[pl.ds(q * L, L)] = outs[q]
        pltpu.sync_copy(out_v, out_hbm)


def kernel(output, label):
    return _topk_sc(_tc_losses(output, label))
